# Initial kernel scaffold; baseline (speedup 1.0000x reference)
import jax
import jax.numpy as jnp
from jax.experimental import pallas as pl


def kernel(x, edge_index, W1, b1, W2, b2, W3, b3, fc_w, fc_b):
    raise NotImplementedError("write your pallas kernel here")


# speedup vs baseline: 19.3997x; 19.3997x over previous
import functools

import jax
import jax.numpy as jnp
from jax import lax
from jax.experimental import pallas as pl
from jax.experimental.pallas import tpu as pltpu
from jax.experimental.pallas import tpu_sc as plsc

N = 100000
E = 3200000
IN_F = 16
HID = 64
OUT_F = 8

NC = 2
NS = 16
LW = 16
SUB = 128
WIN_ROWS = 10
WIN = SUB * WIN_ROWS
N_WIN = 80
EDGES_PER_TILE = WIN * N_WIN
E_PAD = EDGES_PER_TILE * NC * NS
PAD = E_PAD - E
DUMP = 352
N_PAD = N + DUMP
ROWS_PER_TILE_IDX = EDGES_PER_TILE // SUB
SLICE = N_PAD // NS

BR = 2000
NBLK = N // BR

SEGS = [(k * WIN, WIN) for k in range(SLICE // WIN)]
if SLICE % WIN:
  SEGS.append((SLICE // WIN * WIN, SLICE % WIN))


def _scatter_sc_body(n_chunks, src_hbm, dst_hbm, u_hbm, out_hbm,
                     src_v, dst_v, rows_v, agg_sh, sem_g, sem_s):
  c = lax.axis_index("c")
  s = lax.axis_index("s")
  row_base = (c * NS + s) * ROWS_PER_TILE_IDX

  for ch in range(n_chunks):
    def _z(i, _):
      rows_v[i, :] = jnp.zeros((LW,), jnp.float32)
      return 0
    lax.fori_loop(0, WIN, _z, 0)
    zb = s * SLICE
    for off, sz in SEGS:
      pltpu.sync_copy(rows_v.at[pl.ds(0, sz)], agg_sh.at[pl.ds(zb + off, sz)])
    plsc.subcore_barrier()

    def _window(w, _):
      r0 = row_base + w * WIN_ROWS
      pltpu.sync_copy(src_hbm.at[pl.ds(r0, WIN_ROWS)], src_v)
      g = [pltpu.make_async_copy(u_hbm.at[ch].at[src_v.at[j]],
                                 rows_v.at[pl.ds(j * SUB, SUB)], sem_g)
           for j in range(WIN_ROWS)]
      for cp in g:
        cp.start()
      for cp in g:
        cp.wait()
      pltpu.sync_copy(dst_hbm.at[pl.ds(r0, WIN_ROWS)], dst_v)
      sc = [pltpu.make_async_copy(rows_v.at[pl.ds(j * SUB, SUB)],
                                  agg_sh.at[dst_v.at[j]], sem_s)
            for j in range(WIN_ROWS)]
      for cp in sc:
        cp.start(add=True)
      for cp in sc:
        cp.wait()
      return 0
    lax.fori_loop(0, N_WIN, _window, 0)
    plsc.subcore_barrier()

    fb = s * SLICE
    for off, sz in SEGS:
      pltpu.sync_copy(agg_sh.at[pl.ds(fb + off, sz)],
                      out_hbm.at[ch, c, pl.ds(fb + off, sz)])


def _make_scatter_kernel(n_chunks):
  mesh = plsc.VectorSubcoreMesh(core_axis_name="c", subcore_axis_name="s")
  return functools.partial(
      pl.kernel,
      out_type=jax.ShapeDtypeStruct((n_chunks, NC, N_PAD, LW), jnp.float32),
      mesh=mesh,
      scratch_types=[
          pltpu.VMEM((WIN_ROWS, SUB), jnp.int32),
          pltpu.VMEM((WIN_ROWS, SUB), jnp.int32),
          pltpu.VMEM((WIN, LW), jnp.float32),
          pltpu.VMEM_SHARED((N_PAD, LW), jnp.float32),
          pltpu.SemaphoreType.DMA,
          pltpu.SemaphoreType.DMA,
      ],
      compiler_params=pltpu.CompilerParams(use_tc_tiling_on_sc=False),
  )(functools.partial(_scatter_sc_body, n_chunks))


_scatter_c1 = _make_scatter_kernel(1)
_scatter_c4 = _make_scatter_kernel(4)


def _deg_sc_body(dst_hbm, out_hbm, dst_v, ones_v, zeros_v, deg_sh, sem_s):
  c = lax.axis_index("c")
  s = lax.axis_index("s")
  row_base = (c * NS + s) * ROWS_PER_TILE_IDX

  def _o(i, _):
    ones_v[pl.ds(i * LW, LW)] = jnp.ones((LW,), jnp.float32)
    return 0
  lax.fori_loop(0, SUB // LW, _o, 0)

  def _z(i, _):
    zeros_v[pl.ds(i * LW, LW)] = jnp.zeros((LW,), jnp.float32)
    return 0
  lax.fori_loop(0, WIN // LW, _z, 0)

  zb = s * SLICE
  for off, sz in SEGS:
    pltpu.sync_copy(zeros_v.at[pl.ds(0, sz)], deg_sh.at[pl.ds(zb + off, sz)])
  plsc.subcore_barrier()

  def _window(w, _):
    r0 = row_base + w * WIN_ROWS
    pltpu.sync_copy(dst_hbm.at[pl.ds(r0, WIN_ROWS)], dst_v)
    sc = [pltpu.make_async_copy(ones_v, deg_sh.at[dst_v.at[j]], sem_s)
          for j in range(WIN_ROWS)]
    for cp in sc:
      cp.start(add=True)
    for cp in sc:
      cp.wait()
    return 0
  lax.fori_loop(0, N_WIN, _window, 0)
  plsc.subcore_barrier()

  fb = s * SLICE
  for off, sz in SEGS:
    pltpu.sync_copy(deg_sh.at[pl.ds(fb + off, sz)],
                    out_hbm.at[c, 0, pl.ds(fb + off, sz)])


_deg_kernel = pl.kernel(
    _deg_sc_body,
    out_type=jax.ShapeDtypeStruct((NC, 1, N_PAD), jnp.float32),
    mesh=plsc.VectorSubcoreMesh(core_axis_name="c", subcore_axis_name="s"),
    scratch_types=[
        pltpu.VMEM((WIN_ROWS, SUB), jnp.int32),
        pltpu.VMEM((SUB,), jnp.float32),
        pltpu.VMEM((WIN,), jnp.float32),
        pltpu.VMEM_SHARED((N_PAD,), jnp.float32),
        pltpu.SemaphoreType.DMA,
    ],
    compiler_params=pltpu.CompilerParams(use_tc_tiling_on_sc=False),
)


def _t1_body(x_ref, degp_ref, norm_ref, u_ref):
  deg = degp_ref[0] + degp_ref[1]
  nrm = lax.rsqrt(jnp.maximum(deg, 1.0))
  norm_ref[...] = nrm
  u_ref[0] = x_ref[...] * nrm


def _t1_call(x, degp3):
  return pl.pallas_call(
      _t1_body,
      grid=(NBLK,),
      in_specs=[
          pl.BlockSpec((BR, IN_F), lambda i: (i, 0)),
          pl.BlockSpec((2, BR, 1), lambda i: (0, i, 0)),
      ],
      out_specs=[
          pl.BlockSpec((BR, 1), lambda i: (i, 0)),
          pl.BlockSpec((1, BR, LW), lambda i: (0, i, 0)),
      ],
      out_shape=[
          jax.ShapeDtypeStruct((N, 1), jnp.float32),
          jax.ShapeDtypeStruct((1, N, LW), jnp.float32),
      ],
  )(x, degp3)


def _t2_body(n_chunks, p_ref, norm_ref, tx_ref, u2_ref):
  nrm = norm_ref[...]
  for c in range(n_chunks):
    t = -(nrm * (p_ref[c, 0] + p_ref[c, 1]))
    tx_ref[:, c * LW:(c + 1) * LW] = t
    u2_ref[c] = nrm * t


def _t2_call(n_chunks, p, norm):
  f = n_chunks * LW
  return pl.pallas_call(
      functools.partial(_t2_body, n_chunks),
      grid=(NBLK,),
      in_specs=[
          pl.BlockSpec((n_chunks, 2, BR, LW), lambda i: (0, 0, i, 0)),
          pl.BlockSpec((BR, 1), lambda i: (i, 0)),
      ],
      out_specs=[
          pl.BlockSpec((BR, f), lambda i: (i, 0)),
          pl.BlockSpec((n_chunks, BR, LW), lambda i: (0, i, 0)),
      ],
      out_shape=[
          jax.ShapeDtypeStruct((N, f), jnp.float32),
          jax.ShapeDtypeStruct((n_chunks, N, LW), jnp.float32),
      ],
  )(p, norm)


def _cheb_block(n_chunks, h_ref, tx_ref, p2_ref, norm_ref, w_ref, b_ref):
  nrm = norm_ref[...]
  z = jnp.dot(h_ref[...], w_ref[0] - w_ref[2],
              preferred_element_type=jnp.float32)
  z += jnp.dot(tx_ref[...], w_ref[1], preferred_element_type=jnp.float32)
  for c in range(n_chunks):
    m = nrm * (p2_ref[c, 0] + p2_ref[c, 1])
    z -= 2.0 * jnp.dot(m, w_ref[2, c * LW:(c + 1) * LW, :],
                       preferred_element_type=jnp.float32)
  return jnp.maximum(z + b_ref[...], 0.0)


def _t3_body(n_chunks, h_ref, tx_ref, p2_ref, norm_ref, w_ref, b_ref,
             h_out_ref, u_out_ref):
  hn = _cheb_block(n_chunks, h_ref, tx_ref, p2_ref, norm_ref, w_ref, b_ref)
  h_out_ref[...] = hn
  nrm = norm_ref[...]
  for c in range(HID // LW):
    u_out_ref[c] = nrm * hn[:, c * LW:(c + 1) * LW]


def _t3_call(n_chunks, h, tx, p2, norm, w, b2d):
  f_in = n_chunks * LW
  return pl.pallas_call(
      functools.partial(_t3_body, n_chunks),
      grid=(NBLK,),
      in_specs=[
          pl.BlockSpec((BR, f_in), lambda i: (i, 0)),
          pl.BlockSpec((BR, f_in), lambda i: (i, 0)),
          pl.BlockSpec((n_chunks, 2, BR, LW), lambda i: (0, 0, i, 0)),
          pl.BlockSpec((BR, 1), lambda i: (i, 0)),
          pl.BlockSpec((3, f_in, HID), lambda i: (0, 0, 0)),
          pl.BlockSpec((1, HID), lambda i: (0, 0)),
      ],
      out_specs=[
          pl.BlockSpec((BR, HID), lambda i: (i, 0)),
          pl.BlockSpec((HID // LW, BR, LW), lambda i: (0, i, 0)),
      ],
      out_shape=[
          jax.ShapeDtypeStruct((N, HID), jnp.float32),
          jax.ShapeDtypeStruct((HID // LW, N, LW), jnp.float32),
      ],
  )(h, tx, p2, norm, w, b2d)


def _t3f_body(h_ref, tx_ref, p2_ref, norm_ref, w_ref, b_ref,
              fcw_ref, fcb_ref, hsum_ref, logits_ref):
  i = pl.program_id(0)
  hn = _cheb_block(HID // LW, h_ref, tx_ref, p2_ref, norm_ref, w_ref, b_ref)

  @pl.when(i == 0)
  def _():
    hsum_ref[...] = jnp.zeros_like(hsum_ref)

  hsum_ref[...] += jnp.sum(hn, axis=0, keepdims=True)

  @pl.when(i == NBLK - 1)
  def _():
    hg = hsum_ref[...] * (1.0 / N)
    logits_ref[...] = jnp.dot(hg, fcw_ref[...],
                              preferred_element_type=jnp.float32) + fcb_ref[...]


def _t3f_call(h, tx, p2, norm, w, b2d, fc_w, fcb2d):
  _, logits = pl.pallas_call(
      _t3f_body,
      grid=(NBLK,),
      in_specs=[
          pl.BlockSpec((BR, HID), lambda i: (i, 0)),
          pl.BlockSpec((BR, HID), lambda i: (i, 0)),
          pl.BlockSpec((HID // LW, 2, BR, LW), lambda i: (0, 0, i, 0)),
          pl.BlockSpec((BR, 1), lambda i: (i, 0)),
          pl.BlockSpec((3, HID, HID), lambda i: (0, 0, 0)),
          pl.BlockSpec((1, HID), lambda i: (0, 0)),
          pl.BlockSpec((HID, OUT_F), lambda i: (0, 0)),
          pl.BlockSpec((1, OUT_F), lambda i: (0, 0)),
      ],
      out_specs=[
          pl.BlockSpec((1, HID), lambda i: (0, 0)),
          pl.BlockSpec((1, OUT_F), lambda i: (0, 0)),
      ],
      out_shape=[
          jax.ShapeDtypeStruct((1, HID), jnp.float32),
          jax.ShapeDtypeStruct((1, OUT_F), jnp.float32),
      ],
  )(h, tx, p2, norm, w, b2d, fc_w, fcb2d)
  return logits


@jax.jit
def kernel(x, edge_index, W1, b1, W2, b2, W3, b3, fc_w, fc_b):
  src = edge_index[0]
  dst = edge_index[1]
  pad_ids = lax.iota(jnp.int32, PAD)
  src_p = jnp.concatenate([src, pad_ids % 512]).reshape(E_PAD // SUB, SUB)
  dst_p = jnp.concatenate([dst, N + pad_ids % DUMP]).reshape(E_PAD // SUB, SUB)

  degp = _deg_kernel(dst_p)
  degp3 = degp.reshape(NC, N_PAD)[:, :N].reshape(NC, N, 1)
  norm, u = _t1_call(x, degp3)

  h = x
  b2ds = (b1.reshape(1, HID), b2.reshape(1, HID), b3.reshape(1, HID))
  ws = (W1, W2, W3)
  for layer in range(3):
    n_chunks = 1 if layer == 0 else HID // LW
    scat = _scatter_c1 if n_chunks == 1 else _scatter_c4
    p1 = scat(src_p, dst_p, u)
    tx, u2 = _t2_call(n_chunks, p1, norm)
    p2 = scat(src_p, dst_p, u2)
    if layer < 2:
      h, u = _t3_call(n_chunks, h, tx, p2, norm, ws[layer], b2ds[layer])
    else:
      logits = _t3f_call(h, tx, p2, norm, ws[layer], b2ds[layer],
                         fc_w, fc_b.reshape(1, OUT_F))
  return logits


# speedup vs baseline: 20.7681x; 1.0705x over previous
import functools

import jax
import jax.numpy as jnp
from jax import lax
from jax.experimental import pallas as pl
from jax.experimental.pallas import tpu as pltpu
from jax.experimental.pallas import tpu_sc as plsc

N = 100000
E = 3200000
IN_F = 16
HID = 64
OUT_F = 8

NC = 2
NS = 16
LW = 16
SUB = 128
WIN_ROWS = 5
WIN = SUB * WIN_ROWS
N_WIN = 160
EDGES_PER_TILE = WIN * N_WIN
E_PAD = EDGES_PER_TILE * NC * NS
PAD = E_PAD - E
DUMP = 352
N_PAD = N + DUMP
ROWS_PER_TILE_IDX = EDGES_PER_TILE // SUB
SLICE = N_PAD // NS

BR = 2000
NBLK = N // BR

SEGS = [(k * WIN, WIN) for k in range(SLICE // WIN)]
if SLICE % WIN:
  SEGS.append((SLICE // WIN * WIN, SLICE % WIN))


def _scatter_sc_body(n_chunks, src_hbm, dst_hbm, u_hbm, out_hbm,
                     src_v0, src_v1, dst_v0, dst_v1, rows_v0, rows_v1, agg_sh,
                     sem_g0, sem_g1, sem_s0, sem_s1):
  c = lax.axis_index("c")
  s = lax.axis_index("s")
  row_base = (c * NS + s) * ROWS_PER_TILE_IDX
  src_v = (src_v0, src_v1)
  dst_v = (dst_v0, dst_v1)
  rows_v = (rows_v0, rows_v1)
  sem_g = (sem_g0, sem_g1)
  sem_s = (sem_s0, sem_s1)

  def _gathers(ch, b):
    return [pltpu.make_async_copy(u_hbm.at[ch].at[src_v[b].at[j]],
                                  rows_v[b].at[pl.ds(j * SUB, SUB)], sem_g[b])
            for j in range(WIN_ROWS)]

  def _scatters(b):
    return [pltpu.make_async_copy(rows_v[b].at[pl.ds(j * SUB, SUB)],
                                  agg_sh.at[dst_v[b].at[j]], sem_s[b])
            for j in range(WIN_ROWS)]

  def _load_idx(b, w):
    r0 = row_base + w * WIN_ROWS
    pltpu.sync_copy(src_hbm.at[pl.ds(r0, WIN_ROWS)], src_v[b])
    pltpu.sync_copy(dst_hbm.at[pl.ds(r0, WIN_ROWS)], dst_v[b])

  for ch in range(n_chunks):
    def _z(i, _):
      rows_v0[i, :] = jnp.zeros((LW,), jnp.float32)
      return 0
    lax.fori_loop(0, WIN, _z, 0)
    zb = s * SLICE
    for off, sz in SEGS:
      pltpu.sync_copy(rows_v0.at[pl.ds(0, sz)], agg_sh.at[pl.ds(zb + off, sz)])
    plsc.subcore_barrier()

    for b in range(2):
      _load_idx(b, b)
      for cp in _gathers(ch, b):
        cp.start()

    def _pair(k, _):
      for b in range(2):
        for cp in _gathers(ch, b):
          cp.wait()
        for cp in _scatters(b):
          cp.start(add=True)
      for b in range(2):
        w = 2 * k + 2 + b

        @pl.when(w < N_WIN)
        def _():
          for cp in _scatters(b):
            cp.wait()
          _load_idx(b, w)
          for cp in _gathers(ch, b):
            cp.start()
      return 0
    lax.fori_loop(0, N_WIN // 2, _pair, 0)
    for b in range(2):
      for cp in _scatters(b):
        cp.wait()
    plsc.subcore_barrier()

    fb = s * SLICE
    for off, sz in SEGS:
      pltpu.sync_copy(agg_sh.at[pl.ds(fb + off, sz)],
                      out_hbm.at[ch, c, pl.ds(fb + off, sz)])


def _make_scatter_kernel(n_chunks):
  mesh = plsc.VectorSubcoreMesh(core_axis_name="c", subcore_axis_name="s")
  return functools.partial(
      pl.kernel,
      out_type=jax.ShapeDtypeStruct((n_chunks, NC, N_PAD, LW), jnp.float32),
      mesh=mesh,
      scratch_types=[
          pltpu.VMEM((WIN_ROWS, SUB), jnp.int32),
          pltpu.VMEM((WIN_ROWS, SUB), jnp.int32),
          pltpu.VMEM((WIN_ROWS, SUB), jnp.int32),
          pltpu.VMEM((WIN_ROWS, SUB), jnp.int32),
          pltpu.VMEM((WIN, LW), jnp.float32),
          pltpu.VMEM((WIN, LW), jnp.float32),
          pltpu.VMEM_SHARED((N_PAD, LW), jnp.float32),
          pltpu.SemaphoreType.DMA,
          pltpu.SemaphoreType.DMA,
          pltpu.SemaphoreType.DMA,
          pltpu.SemaphoreType.DMA,
      ],
      compiler_params=pltpu.CompilerParams(use_tc_tiling_on_sc=False),
  )(functools.partial(_scatter_sc_body, n_chunks))


_scatter_c1 = _make_scatter_kernel(1)
_scatter_c4 = _make_scatter_kernel(4)


def _deg_sc_body(dst_hbm, out_hbm, dst_v0, dst_v1, ones_v, zeros_v, deg_sh,
                 sem_s0, sem_s1):
  c = lax.axis_index("c")
  s = lax.axis_index("s")
  row_base = (c * NS + s) * ROWS_PER_TILE_IDX
  dst_v = (dst_v0, dst_v1)
  sem_s = (sem_s0, sem_s1)

  def _o(i, _):
    ones_v[pl.ds(i * LW, LW)] = jnp.ones((LW,), jnp.float32)
    return 0
  lax.fori_loop(0, SUB // LW, _o, 0)

  def _z(i, _):
    zeros_v[pl.ds(i * LW, LW)] = jnp.zeros((LW,), jnp.float32)
    return 0
  lax.fori_loop(0, WIN // LW, _z, 0)

  zb = s * SLICE
  for off, sz in SEGS:
    pltpu.sync_copy(zeros_v.at[pl.ds(0, sz)], deg_sh.at[pl.ds(zb + off, sz)])
  plsc.subcore_barrier()

  def _scatters(b):
    return [pltpu.make_async_copy(ones_v, deg_sh.at[dst_v[b].at[j]], sem_s[b])
            for j in range(WIN_ROWS)]

  def _load_idx(b, w):
    r0 = row_base + w * WIN_ROWS
    pltpu.sync_copy(dst_hbm.at[pl.ds(r0, WIN_ROWS)], dst_v[b])

  for b in range(2):
    _load_idx(b, b)

  def _pair(k, _):
    for b in range(2):
      for cp in _scatters(b):
        cp.start(add=True)
    for b in range(2):
      w = 2 * k + 2 + b

      @pl.when(w < N_WIN)
      def _():
        for cp in _scatters(b):
          cp.wait()
        _load_idx(b, w)
    return 0
  lax.fori_loop(0, N_WIN // 2, _pair, 0)
  for b in range(2):
    for cp in _scatters(b):
      cp.wait()
  plsc.subcore_barrier()

  fb = s * SLICE
  for off, sz in SEGS:
    pltpu.sync_copy(deg_sh.at[pl.ds(fb + off, sz)],
                    out_hbm.at[c, 0, pl.ds(fb + off, sz)])


_deg_kernel = pl.kernel(
    _deg_sc_body,
    out_type=jax.ShapeDtypeStruct((NC, 1, N_PAD), jnp.float32),
    mesh=plsc.VectorSubcoreMesh(core_axis_name="c", subcore_axis_name="s"),
    scratch_types=[
        pltpu.VMEM((WIN_ROWS, SUB), jnp.int32),
        pltpu.VMEM((WIN_ROWS, SUB), jnp.int32),
        pltpu.VMEM((SUB,), jnp.float32),
        pltpu.VMEM((WIN,), jnp.float32),
        pltpu.VMEM_SHARED((N_PAD,), jnp.float32),
        pltpu.SemaphoreType.DMA,
        pltpu.SemaphoreType.DMA,
    ],
    compiler_params=pltpu.CompilerParams(use_tc_tiling_on_sc=False),
)


def _t1_body(x_ref, degp_ref, norm_ref, u_ref):
  deg = degp_ref[0] + degp_ref[1]
  nrm = lax.rsqrt(jnp.maximum(deg, 1.0))
  norm_ref[...] = nrm
  u_ref[0] = x_ref[...] * nrm


def _t1_call(x, degp3):
  return pl.pallas_call(
      _t1_body,
      grid=(NBLK,),
      in_specs=[
          pl.BlockSpec((BR, IN_F), lambda i: (i, 0)),
          pl.BlockSpec((2, BR, 1), lambda i: (0, i, 0)),
      ],
      out_specs=[
          pl.BlockSpec((BR, 1), lambda i: (i, 0)),
          pl.BlockSpec((1, BR, LW), lambda i: (0, i, 0)),
      ],
      out_shape=[
          jax.ShapeDtypeStruct((N, 1), jnp.float32),
          jax.ShapeDtypeStruct((1, N, LW), jnp.float32),
      ],
  )(x, degp3)


def _t2_body(n_chunks, p_ref, norm_ref, tx_ref, u2_ref):
  nrm = norm_ref[...]
  for c in range(n_chunks):
    t = -(nrm * (p_ref[c, 0] + p_ref[c, 1]))
    tx_ref[:, c * LW:(c + 1) * LW] = t
    u2_ref[c] = nrm * t


def _t2_call(n_chunks, p, norm):
  f = n_chunks * LW
  return pl.pallas_call(
      functools.partial(_t2_body, n_chunks),
      grid=(NBLK,),
      in_specs=[
          pl.BlockSpec((n_chunks, 2, BR, LW), lambda i: (0, 0, i, 0)),
          pl.BlockSpec((BR, 1), lambda i: (i, 0)),
      ],
      out_specs=[
          pl.BlockSpec((BR, f), lambda i: (i, 0)),
          pl.BlockSpec((n_chunks, BR, LW), lambda i: (0, i, 0)),
      ],
      out_shape=[
          jax.ShapeDtypeStruct((N, f), jnp.float32),
          jax.ShapeDtypeStruct((n_chunks, N, LW), jnp.float32),
      ],
  )(p, norm)


def _cheb_block(n_chunks, h_ref, tx_ref, p2_ref, norm_ref, w_ref, b_ref):
  nrm = norm_ref[...]
  z = jnp.dot(h_ref[...], w_ref[0] - w_ref[2],
              preferred_element_type=jnp.float32)
  z += jnp.dot(tx_ref[...], w_ref[1], preferred_element_type=jnp.float32)
  for c in range(n_chunks):
    m = nrm * (p2_ref[c, 0] + p2_ref[c, 1])
    z -= 2.0 * jnp.dot(m, w_ref[2, c * LW:(c + 1) * LW, :],
                       preferred_element_type=jnp.float32)
  return jnp.maximum(z + b_ref[...], 0.0)


def _t3_body(n_chunks, h_ref, tx_ref, p2_ref, norm_ref, w_ref, b_ref,
             h_out_ref, u_out_ref):
  hn = _cheb_block(n_chunks, h_ref, tx_ref, p2_ref, norm_ref, w_ref, b_ref)
  h_out_ref[...] = hn
  nrm = norm_ref[...]
  for c in range(HID // LW):
    u_out_ref[c] = nrm * hn[:, c * LW:(c + 1) * LW]


def _t3_call(n_chunks, h, tx, p2, norm, w, b2d):
  f_in = n_chunks * LW
  return pl.pallas_call(
      functools.partial(_t3_body, n_chunks),
      grid=(NBLK,),
      in_specs=[
          pl.BlockSpec((BR, f_in), lambda i: (i, 0)),
          pl.BlockSpec((BR, f_in), lambda i: (i, 0)),
          pl.BlockSpec((n_chunks, 2, BR, LW), lambda i: (0, 0, i, 0)),
          pl.BlockSpec((BR, 1), lambda i: (i, 0)),
          pl.BlockSpec((3, f_in, HID), lambda i: (0, 0, 0)),
          pl.BlockSpec((1, HID), lambda i: (0, 0)),
      ],
      out_specs=[
          pl.BlockSpec((BR, HID), lambda i: (i, 0)),
          pl.BlockSpec((HID // LW, BR, LW), lambda i: (0, i, 0)),
      ],
      out_shape=[
          jax.ShapeDtypeStruct((N, HID), jnp.float32),
          jax.ShapeDtypeStruct((HID // LW, N, LW), jnp.float32),
      ],
  )(h, tx, p2, norm, w, b2d)


def _t3f_body(h_ref, tx_ref, p2_ref, norm_ref, w_ref, b_ref,
              fcw_ref, fcb_ref, hsum_ref, logits_ref):
  i = pl.program_id(0)
  hn = _cheb_block(HID // LW, h_ref, tx_ref, p2_ref, norm_ref, w_ref, b_ref)

  @pl.when(i == 0)
  def _():
    hsum_ref[...] = jnp.zeros_like(hsum_ref)

  hsum_ref[...] += jnp.sum(hn, axis=0, keepdims=True)

  @pl.when(i == NBLK - 1)
  def _():
    hg = hsum_ref[...] * (1.0 / N)
    logits_ref[...] = jnp.dot(hg, fcw_ref[...],
                              preferred_element_type=jnp.float32) + fcb_ref[...]


def _t3f_call(h, tx, p2, norm, w, b2d, fc_w, fcb2d):
  _, logits = pl.pallas_call(
      _t3f_body,
      grid=(NBLK,),
      in_specs=[
          pl.BlockSpec((BR, HID), lambda i: (i, 0)),
          pl.BlockSpec((BR, HID), lambda i: (i, 0)),
          pl.BlockSpec((HID // LW, 2, BR, LW), lambda i: (0, 0, i, 0)),
          pl.BlockSpec((BR, 1), lambda i: (i, 0)),
          pl.BlockSpec((3, HID, HID), lambda i: (0, 0, 0)),
          pl.BlockSpec((1, HID), lambda i: (0, 0)),
          pl.BlockSpec((HID, OUT_F), lambda i: (0, 0)),
          pl.BlockSpec((1, OUT_F), lambda i: (0, 0)),
      ],
      out_specs=[
          pl.BlockSpec((1, HID), lambda i: (0, 0)),
          pl.BlockSpec((1, OUT_F), lambda i: (0, 0)),
      ],
      out_shape=[
          jax.ShapeDtypeStruct((1, HID), jnp.float32),
          jax.ShapeDtypeStruct((1, OUT_F), jnp.float32),
      ],
  )(h, tx, p2, norm, w, b2d, fc_w, fcb2d)
  return logits


@jax.jit
def kernel(x, edge_index, W1, b1, W2, b2, W3, b3, fc_w, fc_b):
  src = edge_index[0]
  dst = edge_index[1]
  pad_ids = lax.iota(jnp.int32, PAD)
  src_p = jnp.concatenate([src, pad_ids % 512]).reshape(E_PAD // SUB, SUB)
  dst_p = jnp.concatenate([dst, N + pad_ids % DUMP]).reshape(E_PAD // SUB, SUB)

  degp = _deg_kernel(dst_p)
  degp3 = degp.reshape(NC, N_PAD)[:, :N].reshape(NC, N, 1)
  norm, u = _t1_call(x, degp3)

  h = x
  b2ds = (b1.reshape(1, HID), b2.reshape(1, HID), b3.reshape(1, HID))
  ws = (W1, W2, W3)
  for layer in range(3):
    n_chunks = 1 if layer == 0 else HID // LW
    scat = _scatter_c1 if n_chunks == 1 else _scatter_c4
    p1 = scat(src_p, dst_p, u)
    tx, u2 = _t2_call(n_chunks, p1, norm)
    p2 = scat(src_p, dst_p, u2)
    if layer < 2:
      h, u = _t3_call(n_chunks, h, tx, p2, norm, ws[layer], b2ds[layer])
    else:
      logits = _t3f_call(h, tx, p2, norm, ws[layer], b2ds[layer],
                         fc_w, fc_b.reshape(1, OUT_F))
  return logits
